# baseline (speedup 1.0000x reference)
import functools

import jax
import jax.numpy as jnp
from jax import lax
from jax.experimental import pallas as pl
from jax.experimental.pallas import tpu as pltpu
from jax.experimental.pallas import tpu_sc as plsc

EMB = 64
NW = 32
CH = 128


def _make_embed(B: int):
    b_per_w = B // NW
    nch = b_per_w // CH
    mesh = plsc.VectorSubcoreMesh(core_axis_name="c", subcore_axis_name="s")

    @functools.partial(
        pl.kernel,
        mesh=mesh,
        out_type=jax.ShapeDtypeStruct((B, EMB), jnp.float32),
        scratch_types=[
            pltpu.VMEM((nch, CH), jnp.int32),
            pltpu.VMEM((CH, EMB), jnp.float32),
            pltpu.SemaphoreType.DMA,
        ],
        compiler_params=pltpu.CompilerParams(use_tc_tiling_on_sc=False),
    )
    def embed(words_hbm, table_hbm, out_hbm, idx_v, rows_v, sem):
        wid = lax.axis_index("s") * 2 + lax.axis_index("c")
        base = wid * b_per_w
        pltpu.sync_copy(words_hbm.at[wid], idx_v)

        def body(c, carry):
            pltpu.async_copy(table_hbm.at[idx_v.at[c]], rows_v, sem).wait()
            pltpu.sync_copy(rows_v, out_hbm.at[pl.ds(base + c * CH, CH)])
            return carry

        lax.fori_loop(0, nch, body, 0)

    return embed


def kernel(words, weight):
    B = words.shape[0] * words.shape[1]
    idx = words.reshape(NW, B // NW // CH, CH)
    out = _make_embed(B)(idx, weight)
    return out.reshape(words.shape[0], words.shape[1], EMB)


# speedup vs baseline: 1.0452x; 1.0452x over previous
import functools

import jax
import jax.numpy as jnp
from jax import lax
from jax.experimental import pallas as pl
from jax.experimental.pallas import tpu as pltpu
from jax.experimental.pallas import tpu_sc as plsc

EMB = 64
NW = 32
CH = 128
NBUF = 10


def _make_embed(B: int):
    b_per_w = B // NW
    nch = b_per_w // CH
    ngroups = nch // NBUF
    mesh = plsc.VectorSubcoreMesh(core_axis_name="c", subcore_axis_name="s")

    scratch = [pltpu.VMEM((nch, CH), jnp.int32)]
    scratch += [pltpu.VMEM((CH, EMB), jnp.float32) for _ in range(NBUF)]
    scratch += [pltpu.SemaphoreType.DMA for _ in range(2 * NBUF)]

    @functools.partial(
        pl.kernel,
        mesh=mesh,
        out_type=jax.ShapeDtypeStruct((B, EMB), jnp.float32),
        scratch_types=scratch,
        compiler_params=pltpu.CompilerParams(use_tc_tiling_on_sc=False),
    )
    def embed(words_hbm, table_hbm, out_hbm, idx_v, *bufs):
        rows = bufs[:NBUF]
        gsem = bufs[NBUF:2 * NBUF]
        osem = bufs[2 * NBUF:]
        wid = lax.axis_index("s") * 2 + lax.axis_index("c")
        base = wid * b_per_w
        pltpu.sync_copy(words_hbm.at[wid], idx_v)

        def fire_gather(c, b):
            pltpu.async_copy(table_hbm.at[idx_v.at[c]], rows[b], gsem[b])

        def wait_gather(b):
            pltpu.make_async_copy(
                table_hbm.at[idx_v.at[0]], rows[b], gsem[b]).wait()

        def fire_out(c, b):
            pltpu.async_copy(
                rows[b], out_hbm.at[pl.ds(base + c * CH, CH)], osem[b])

        def wait_out(b):
            pltpu.make_async_copy(
                rows[b], out_hbm.at[pl.ds(base, CH)], osem[b]).wait()

        for b in range(NBUF):
            fire_gather(b, b)

        def group(g, carry):
            c0 = g * NBUF
            for b in range(NBUF):
                wait_gather(b)
                fire_out(c0 + b, b)
            for b in range(NBUF):
                wait_out(b)
                fire_gather(c0 + NBUF + b, b)
            return carry

        lax.fori_loop(0, ngroups - 1, group, 0)

        c0 = (ngroups - 1) * NBUF
        for b in range(NBUF):
            wait_gather(b)
            fire_out(c0 + b, b)
        for b in range(NBUF):
            wait_out(b)

    return embed


def kernel(words, weight):
    B = words.shape[0] * words.shape[1]
    idx = words.reshape(NW, B // NW // CH, CH)
    out = _make_embed(B)(idx, weight)
    return out.reshape(words.shape[0], words.shape[1], EMB)


# speedup vs baseline: 1.0609x; 1.0150x over previous
import functools

import jax
import jax.numpy as jnp
from jax import lax
from jax.experimental import pallas as pl
from jax.experimental.pallas import tpu as pltpu
from jax.experimental.pallas import tpu_sc as plsc

EMB = 64
NW = 32
CH = 128
NBUF = 10


def _make_embed(n_seq: int, n_batch: int):
    n_chunks = n_seq * (n_batch // CH)
    cpw = n_chunks // NW
    ngroups = cpw // NBUF
    bpr = n_batch // CH
    mesh = plsc.VectorSubcoreMesh(core_axis_name="c", subcore_axis_name="s")

    scratch = [pltpu.VMEM((cpw, CH), jnp.int32)]
    scratch += [pltpu.VMEM((CH, EMB), jnp.float32) for _ in range(NBUF)]
    scratch += [pltpu.SemaphoreType.DMA for _ in range(2 * NBUF + 1)]

    @functools.partial(
        pl.kernel,
        mesh=mesh,
        out_type=jax.ShapeDtypeStruct((n_seq, n_batch, EMB), jnp.float32),
        scratch_types=scratch,
        compiler_params=pltpu.CompilerParams(use_tc_tiling_on_sc=False),
    )
    def embed(wordsT_hbm, table_hbm, out_hbm, idx_v, *bufs):
        rows = bufs[:NBUF]
        gsem = bufs[NBUF:2 * NBUF]
        osem = bufs[2 * NBUF:3 * NBUF]
        isem = bufs[3 * NBUF]
        wid = lax.axis_index("s") * 2 + lax.axis_index("c")
        g0 = wid * cpw

        def stage(i, carry):
            g = g0 + i
            pltpu.async_copy(
                wordsT_hbm.at[g // bpr, pl.ds((g % bpr) * CH, CH)],
                idx_v.at[i], isem)
            return carry

        lax.fori_loop(0, cpw, stage, 0)

        def drain_idx(i, carry):
            pltpu.make_async_copy(
                wordsT_hbm.at[0, pl.ds(0, CH)], idx_v.at[i], isem).wait()
            return carry

        lax.fori_loop(0, cpw, drain_idx, 0)

        def fire_gather(c, b):
            pltpu.async_copy(table_hbm.at[idx_v.at[c]], rows[b], gsem[b])

        def wait_gather(b):
            pltpu.make_async_copy(
                table_hbm.at[idx_v.at[0]], rows[b], gsem[b]).wait()

        def fire_out(c, b):
            g = g0 + c
            pltpu.async_copy(
                rows[b],
                out_hbm.at[g // bpr, pl.ds((g % bpr) * CH, CH)],
                osem[b])

        def wait_out(b):
            pltpu.make_async_copy(
                rows[b], out_hbm.at[0, pl.ds(0, CH)], osem[b]).wait()

        for b in range(NBUF):
            fire_gather(b, b)

        def group(g, carry):
            c0 = g * NBUF
            for b in range(NBUF):
                wait_gather(b)
                fire_out(c0 + b, b)
            for b in range(NBUF):
                wait_out(b)
                fire_gather(c0 + NBUF + b, b)
            return carry

        lax.fori_loop(0, ngroups - 1, group, 0)

        c0 = (ngroups - 1) * NBUF
        for b in range(NBUF):
            wait_gather(b)
            fire_out(c0 + b, b)
        for b in range(NBUF):
            wait_out(b)

    return embed


def kernel(words, weight):
    n_batch, n_seq = words.shape
    out = _make_embed(n_seq, n_batch)(words.T, weight)
    return out.transpose(1, 0, 2)


# speedup vs baseline: 2.1282x; 2.0060x over previous
import functools

import jax
import jax.numpy as jnp
from jax import lax
from jax.experimental import pallas as pl
from jax.experimental.pallas import tpu as pltpu
from jax.experimental.pallas import tpu_sc as plsc

EMB = 64
NW = 32
CH = 128
NBUF = 10
VB = 16384
SH = VB.bit_length() - 1


def _transpose_block(wt0_ref, wt1_ref, out_ref):
    out_ref[:, :EMB] = wt0_ref[...].T
    out_ref[:, EMB:] = wt1_ref[...].T


def _make_transpose(vocab: int):
    grid = (vocab + 2 * VB - 1) // (2 * VB)
    last = (vocab - 1) // VB
    return pl.pallas_call(
        _transpose_block,
        grid=(grid,),
        in_specs=[
            pl.BlockSpec((EMB, VB), lambda i: (0, jnp.minimum(2 * i, last))),
            pl.BlockSpec(
                (EMB, VB), lambda i: (0, jnp.minimum(2 * i + 1, last))),
        ],
        out_specs=pl.BlockSpec((VB, 2 * EMB), lambda i: (i, 0)),
        out_shape=jax.ShapeDtypeStruct((grid * VB, 2 * EMB), jnp.float32),
        compiler_params=pltpu.CompilerParams(
            dimension_semantics=("arbitrary",)),
    )


def _make_gather(n_seq: int, n_batch: int, vocab: int):
    n_chunks = n_seq * (n_batch // CH)
    cpw = n_chunks // NW
    ngroups = cpw // NBUF
    bpr = n_batch // CH
    mesh = plsc.VectorSubcoreMesh(core_axis_name="c", subcore_axis_name="s")

    scratch = [pltpu.VMEM((cpw, CH), jnp.int32)]
    scratch += [pltpu.VMEM((CH, EMB), jnp.float32) for _ in range(NBUF)]
    scratch += [pltpu.SemaphoreType.DMA for _ in range(2 * NBUF + 1)]

    @functools.partial(
        pl.kernel,
        mesh=mesh,
        out_type=jax.ShapeDtypeStruct((n_seq, n_batch, EMB), jnp.float32),
        scratch_types=scratch,
        compiler_params=pltpu.CompilerParams(use_tc_tiling_on_sc=False),
    )
    def gather(wordsT_hbm, table_hbm, out_hbm, idx_v, *bufs):
        rows = bufs[:NBUF]
        gsem = bufs[NBUF:2 * NBUF]
        osem = bufs[2 * NBUF:3 * NBUF]
        isem = bufs[3 * NBUF]
        wid = lax.axis_index("s") * 2 + lax.axis_index("c")
        g0 = wid * cpw

        def stage(i, carry):
            g = g0 + i
            pltpu.async_copy(
                wordsT_hbm.at[g // bpr, pl.ds((g % bpr) * CH, CH)],
                idx_v.at[i], isem)
            return carry

        lax.fori_loop(0, cpw, stage, 0)

        def drain_idx(i, carry):
            pltpu.make_async_copy(
                wordsT_hbm.at[0, pl.ds(0, CH)], idx_v.at[i], isem).wait()
            return carry

        lax.fori_loop(0, cpw, drain_idx, 0)

        def fire_gather(c, b):
            pltpu.async_copy(table_hbm.at[idx_v.at[c]], rows[b], gsem[b])

        def wait_gather(b):
            pltpu.make_async_copy(
                table_hbm.at[idx_v.at[0]], rows[b], gsem[b]).wait()

        def fire_out(c, b):
            g = g0 + c
            pltpu.async_copy(
                rows[b],
                out_hbm.at[g // bpr, pl.ds((g % bpr) * CH, CH)],
                osem[b])

        def wait_out(b):
            pltpu.make_async_copy(
                rows[b], out_hbm.at[0, pl.ds(0, CH)], osem[b]).wait()

        for b in range(NBUF):
            fire_gather(b, b)

        def group(g, carry):
            c0 = g * NBUF
            for b in range(NBUF):
                wait_gather(b)
                fire_out(c0 + b, b)
            for b in range(NBUF):
                wait_out(b)
                fire_gather(c0 + NBUF + b, b)
            return carry

        lax.fori_loop(0, ngroups - 1, group, 0)

        c0 = (ngroups - 1) * NBUF
        for b in range(NBUF):
            wait_gather(b)
            fire_out(c0 + b, b)
        for b in range(NBUF):
            wait_out(b)

    return gather


def kernel(words, weight):
    n_batch, n_seq = words.shape
    vocab = weight.shape[0]
    packed = _make_transpose(vocab)(weight.T, weight.T)
    table = packed.reshape(packed.shape[0] * 2, EMB)
    lin = (2 * ((words >> (SH + 1) << SH) | (words & (VB - 1)))
           + ((words >> SH) & 1))
    out = _make_gather(n_seq, n_batch, table.shape[0])(lin.T, table)
    return out.transpose(1, 0, 2)
